# baseline (speedup 1.0000x reference)
import jax
import jax.numpy as jnp
import numpy as np
from jax import lax
from jax.experimental import pallas as pl
from jax.experimental.pallas import tpu as pltpu

H = 1024
B = 128
P = 16
S = 12
REL = 5000
ENT = 20000
N = B * P
TAO = 0.5
BN_SCALE = 1.0 / float(np.sqrt(1.0 + 1e-5))
F32 = jnp.float32

_NB1 = 128
_G1 = N // _NB1
_STR = 136


def _gather_body(idx_ref, rel_hbm, x_ref, rel_v, tile_a, tile_b, sem):
    c = pl.program_id(0)
    i = pl.program_id(1)

    @pl.when(i == 0)
    def _():
        cp = pltpu.make_async_copy(rel_hbm, rel_v, sem)
        cp.start()
        cp.wait()

    base = (c * (_G1 // 2) + i) * (_NB1 * S)
    for s in range(S):
        tile = tile_a if (s % 2 == 0) else tile_b
        for mi in range(_NB1):
            r8 = pl.multiple_of(idx_ref[base + mi * S + s], 8)
            tile[mi: mi + 8 * _STR: _STR, :] = rel_v[pl.ds(r8, 8), :]
        xs = jnp.concatenate(
            [tile[pl.ds(j * _STR, _NB1), :] for j in range(8)], axis=-1)
        x_ref[s] = xs


def _gather_call(idx8, rel2):
    return pl.pallas_call(
        _gather_body,
        grid_spec=pltpu.PrefetchScalarGridSpec(
            num_scalar_prefetch=1,
            grid=(2, _G1 // 2),
            in_specs=[pl.BlockSpec(memory_space=pl.ANY)],
            out_specs=pl.BlockSpec(
                (S, _NB1, H), lambda c, i, idx: (0, c * (_G1 // 2) + i, 0)),
            scratch_shapes=[
                pltpu.VMEM((REL * 8, 128), F32),
                pltpu.VMEM((8 * _STR, 128), F32),
                pltpu.VMEM((8 * _STR, 128), F32),
                pltpu.SemaphoreType.DMA,
            ],
        ),
        out_shape=jax.ShapeDtypeStruct((S, N, H), F32),
        compiler_params=pltpu.CompilerParams(
            dimension_semantics=(pltpu.ARBITRARY, pltpu.ARBITRARY),
            vmem_limit_bytes=52 * 1024 * 1024,
        ),
        name="k1_gather",
    )(idx8, rel2)


_NB2 = 64
_G2 = N // _NB2


def _lstm_body(x_ref, wih_hbm, whh_hbm, b4_ref, hs_ref, wih_v, whh_v, sems):
    i = pl.program_id(1)

    @pl.when(i == 0)
    def _():
        cp1 = pltpu.make_async_copy(wih_hbm, wih_v, sems.at[0])
        cp2 = pltpu.make_async_copy(whh_hbm, whh_v, sems.at[1])
        cp1.start()
        cp2.start()
        cp1.wait()
        cp2.wait()

    b4 = b4_ref[...]
    h = jnp.zeros((_NB2, H), F32)
    cc = jnp.zeros((_NB2, H), F32)
    for s in range(S):
        g = (jnp.dot(x_ref[s], wih_v[...], preferred_element_type=F32)
             + jnp.dot(h, whh_v[...], preferred_element_type=F32) + b4)
        ig = jax.nn.sigmoid(g[:, :H])
        fg = jax.nn.sigmoid(g[:, H:2 * H])
        gg = jnp.tanh(g[:, 2 * H:3 * H])
        og = jax.nn.sigmoid(g[:, 3 * H:])
        cc = fg * cc + ig * gg
        h = og * jnp.tanh(cc)
        hs_ref[s] = h


def _lstm_call(x, Wih, Whh, b4):
    return pl.pallas_call(
        _lstm_body,
        grid=(2, _G2 // 2),
        in_specs=[
            pl.BlockSpec((S, _NB2, H), lambda c, i: (0, c * (_G2 // 2) + i, 0)),
            pl.BlockSpec(memory_space=pl.ANY),
            pl.BlockSpec(memory_space=pl.ANY),
            pl.BlockSpec((1, 4 * H), lambda c, i: (0, 0)),
        ],
        out_specs=pl.BlockSpec(
            (S, _NB2, H), lambda c, i: (0, c * (_G2 // 2) + i, 0)),
        scratch_shapes=[
            pltpu.VMEM((H, 4 * H), F32),
            pltpu.VMEM((H, 4 * H), F32),
            pltpu.SemaphoreType.DMA((2,)),
        ],
        out_shape=jax.ShapeDtypeStruct((S, N, H), F32),
        compiler_params=pltpu.CompilerParams(
            dimension_semantics=(pltpu.ARBITRARY, pltpu.ARBITRARY),
            vmem_limit_bytes=54 * 1024 * 1024,
        ),
        name="k2_lstm",
    )(x, Wih, Whh, b4)


_KB3 = 512
_G3 = 20


def _ent_body(ents_ref, we_ref, out_ref):
    kc = pl.program_id(0)
    ki = pl.program_id(1)
    kabs = kc * _G3 + ki
    valid = ENT - kabs * _KB3
    row = lax.broadcasted_iota(jnp.int32, (_KB3, H + 128), 0)
    w_aug = jnp.concatenate(
        [we_ref[...], jnp.ones((_KB3, 128), F32)], axis=-1)
    w_aug = jnp.where(row < valid, w_aug, 0.0)
    m = ents_ref[...].reshape(N, _KB3).astype(F32)
    part = jnp.dot(m, w_aug, preferred_element_type=F32)

    @pl.when(ki == 0)
    def _():
        out_ref[0] = part

    @pl.when(ki > 0)
    def _():
        out_ref[0] = out_ref[0] + part


def _ent_call(ents2, ent_E):
    return pl.pallas_call(
        _ent_body,
        grid=(2, _G3),
        in_specs=[
            pl.BlockSpec((B, P, _KB3), lambda kc, ki: (0, 0, kc * _G3 + ki)),
            pl.BlockSpec((_KB3, H), lambda kc, ki: (kc * _G3 + ki, 0)),
        ],
        out_specs=pl.BlockSpec((1, N, H + 128), lambda kc, ki: (kc, 0, 0)),
        out_shape=jax.ShapeDtypeStruct((2, N, H + 128), F32),
        compiler_params=pltpu.CompilerParams(
            dimension_semantics=(pltpu.ARBITRARY, pltpu.ARBITRARY),
            vmem_limit_bytes=54 * 1024 * 1024,
        ),
        name="k3_entities",
    )(ents2, ent_E)


_NB4 = 128
_G4 = N // _NB4


def _rec_body(hs_ref, pe_ref, w1a_ref, w1b_ref, w2_ref, b1_ref, b2_ref,
              tree_ref, ep_ref):
    b1 = b1_ref[...]
    b2 = b2_ref[...]
    e = hs_ref[0]
    for t in range(1, S):
        a = (jnp.dot(e, w1a_ref[...], preferred_element_type=F32)
             + jnp.dot(hs_ref[t], w1b_ref[...], preferred_element_type=F32)
             + b1)
        a = jnp.maximum(a, 0.01 * a)
        e = jnp.dot(a, w2_ref[...], preferred_element_type=F32) + b2
        e = jnp.maximum(e, 0.01 * e)
    tree_ref[...] = jnp.mean(e.reshape(_NB4 // P, P, H), axis=1)

    pe = pe_ref[0] + pe_ref[1]
    cnt = pe[:, H:]
    rcp = 1.0 / jnp.maximum(cnt, 1.0)
    ep = pe[:, :H] * pltpu.repeat(rcp, 8, axis=1)
    ep_ref[...] = jnp.mean(ep.reshape(_NB4 // P, P, H), axis=1)


def _rec_call(hs, pe_parts, Wm1a, Wm1b, Wm2, bm1, bm2):
    nb = _NB4 // P
    return pl.pallas_call(
        _rec_body,
        grid=(2, _G4 // 2),
        in_specs=[
            pl.BlockSpec((S, _NB4, H), lambda c, i: (0, c * (_G4 // 2) + i, 0)),
            pl.BlockSpec((2, _NB4, H + 128),
                         lambda c, i: (0, c * (_G4 // 2) + i, 0)),
            pl.BlockSpec((H, H), lambda c, i: (0, 0)),
            pl.BlockSpec((H, H), lambda c, i: (0, 0)),
            pl.BlockSpec((H, H), lambda c, i: (0, 0)),
            pl.BlockSpec((1, H), lambda c, i: (0, 0)),
            pl.BlockSpec((1, H), lambda c, i: (0, 0)),
        ],
        out_specs=[
            pl.BlockSpec((nb, H), lambda c, i: (c * (_G4 // 2) + i, 0)),
            pl.BlockSpec((nb, H), lambda c, i: (c * (_G4 // 2) + i, 0)),
        ],
        out_shape=[
            jax.ShapeDtypeStruct((B, H), F32),
            jax.ShapeDtypeStruct((B, H), F32),
        ],
        compiler_params=pltpu.CompilerParams(
            dimension_semantics=(pltpu.ARBITRARY, pltpu.ARBITRARY),
            vmem_limit_bytes=54 * 1024 * 1024,
        ),
        name="k4_rec",
    )(hs, pe_parts, Wm1a, Wm1b, Wm2, bm1, bm2)


def _ln(x, g, b):
    m = jnp.mean(x, axis=-1, keepdims=True)
    v = jnp.mean((x - m) ** 2, axis=-1, keepdims=True)
    return (x - m) * lax.rsqrt(v + 1e-5) * g + b


def _head_body(tree_ref, ep_ref, wf1, bf1_, g1e_, b1e_, g2e_, b2e_, wf2, bf2_,
               wv, bv_, wo, bo_, g1t_, b1t_, wp1, bp1_, wp2, bp2_,
               loss_ref, p_ref):
    e = _ln(ep_ref[...], g1e_[...], b1e_[...])
    e = jnp.maximum(
        jnp.dot(e, wf1[...], preferred_element_type=F32) + bf1_[...], 0.0
    ) * BN_SCALE
    e = _ln(e, g2e_[...], b2e_[...])
    e = jnp.maximum(
        jnp.dot(e, wf2[...], preferred_element_type=F32) + bf2_[...], 0.0
    ) * BN_SCALE
    v = jnp.dot(e, wv[...], preferred_element_type=F32) + bv_[...]
    attn = jnp.dot(v, wo[...], preferred_element_type=F32) + bo_[...]
    tree = _ln(tree_ref[...] + attn, g1t_[...], b1t_[...])
    ph = jnp.maximum(
        jnp.dot(tree, wp1[...], preferred_element_type=F32) + bp1_[...], 0.0)
    p = jnp.dot(ph, wp2[...], preferred_element_type=F32) + bp2_[...]
    nrm = jnp.sqrt(jnp.sum(p * p, axis=-1, keepdims=True))
    p = p / jnp.maximum(nrm, 1e-12)
    p_ref[...] = p

    sim = lax.dot_general(p, p, (((1,), (1,)), ((), ())),
                          preferred_element_type=F32)
    r = lax.broadcasted_iota(jnp.int32, (B, B), 0)
    cdx = lax.broadcasted_iota(jnp.int32, (B, B), 1)
    sim = (sim - (r == cdx).astype(F32)) * (1.0 / TAO)
    mx = jnp.max(sim, axis=-1, keepdims=True)
    lse = jnp.log(jnp.sum(jnp.exp(sim - mx), axis=-1, keepdims=True)) + mx
    ysel = (cdx == jnp.bitwise_xor(r, 1)).astype(F32)
    picked = jnp.sum(sim * ysel, axis=-1, keepdims=True) - lse
    loss_ref[...] = -jnp.sum(picked, axis=0, keepdims=True) * (1.0 / B)


def _head_call(tree, epre, Wf1, bf1, g1e, b1e, g2e, b2e, Wf2, bf2,
               Wv, bv, Wo, bo, g1t, b1t, Wp1, bp1, Wp2, bp2):
    return pl.pallas_call(
        _head_body,
        out_shape=[
            jax.ShapeDtypeStruct((1, 1), F32),
            jax.ShapeDtypeStruct((B, H), F32),
        ],
        compiler_params=pltpu.CompilerParams(
            vmem_limit_bytes=54 * 1024 * 1024,
        ),
        name="k5_head",
    )(tree, epre, Wf1, bf1, g1e, b1e, g2e, b2e, Wf2, bf2,
      Wv, bv, Wo, bo, g1t, b1t, Wp1, bp1, Wp2, bp2)


def kernel(rel_E, Wih, Whh, bih, bhh, Wm1, bm1, Wm2, bm2,
           ent_E, Wf1, bf1, Wf2, bf2, g1e, b1e, g2e, b2e,
           Wq, bq, Wk, bk, Wv, bv, Wo, bo, g1t, b1t,
           Wp1, bp1, Wp2, bp2, paths, entities):
    r2 = lambda a: a.reshape(1, -1)
    idx8 = paths.reshape(-1).astype(jnp.int32) * 8
    rel2 = rel_E.reshape(REL * 8, 128)
    b4 = r2(bih + bhh)

    x = _gather_call(idx8, rel2)
    hs = _lstm_call(x, Wih, Whh, b4)
    pe_parts = _ent_call(entities, ent_E)
    tree, epre = _rec_call(hs, pe_parts, Wm1[:H], Wm1[H:], Wm2,
                           r2(bm1), r2(bm2))
    loss2, p = _head_call(tree, epre, Wf1, r2(bf1), r2(g1e), r2(b1e),
                          r2(g2e), r2(b2e), Wf2, r2(bf2), Wv, r2(bv),
                          Wo, r2(bo), r2(g1t), r2(b1t), Wp1, r2(bp1),
                          Wp2, r2(bp2))
    return loss2.reshape(()), p


# speedup vs baseline: 1.5376x; 1.5376x over previous
import jax
import jax.numpy as jnp
import numpy as np
from jax import lax
from jax.experimental import pallas as pl
from jax.experimental.pallas import tpu as pltpu

H = 1024
B = 128
P = 16
S = 12
REL = 5000
ENT = 20000
N = B * P
TAO = 0.5
BN_SCALE = 1.0 / float(np.sqrt(1.0 + 1e-5))
F32 = jnp.float32

_NB1 = 128
_G1 = N // _NB1
_STR = 136


def _gather_body(idx_ref, rel_hbm, x_ref, rel_v, tile_a, tile_b, sem):
    c = pl.program_id(0)
    i = pl.program_id(1)

    @pl.when(i == 0)
    def _():
        cp = pltpu.make_async_copy(rel_hbm, rel_v, sem)
        cp.start()
        cp.wait()

    base = (c * (_G1 // 2) + i) * (_NB1 * S)
    for s in range(S):
        tile = tile_a if (s % 2 == 0) else tile_b
        for mi in range(_NB1):
            r8 = pl.multiple_of(idx_ref[base + mi * S + s], 8)
            tile[mi: mi + 8 * _STR: _STR, :] = rel_v[pl.ds(r8, 8), :]
        xs = jnp.concatenate(
            [tile[pl.ds(j * _STR, _NB1), :] for j in range(8)], axis=-1)
        x_ref[s] = xs


def _gather_call(idx8, rel2):
    return pl.pallas_call(
        _gather_body,
        grid_spec=pltpu.PrefetchScalarGridSpec(
            num_scalar_prefetch=1,
            grid=(2, _G1 // 2),
            in_specs=[pl.BlockSpec(memory_space=pl.ANY)],
            out_specs=pl.BlockSpec(
                (S, _NB1, H), lambda c, i, idx: (0, c * (_G1 // 2) + i, 0)),
            scratch_shapes=[
                pltpu.VMEM((REL * 8, 128), F32),
                pltpu.VMEM((8 * _STR, 128), F32),
                pltpu.VMEM((8 * _STR, 128), F32),
                pltpu.SemaphoreType.DMA,
            ],
        ),
        out_shape=jax.ShapeDtypeStruct((S, N, H), F32),
        compiler_params=pltpu.CompilerParams(
            dimension_semantics=(pltpu.ARBITRARY, pltpu.ARBITRARY),
            vmem_limit_bytes=52 * 1024 * 1024,
        ),
        name="k1_gather",
    )(idx8, rel2)


_NB2 = 128
_G2 = N // _NB2
BF16 = jnp.bfloat16


def _lstm_body(x_ref, wih_hbm, whh_hbm, b4_ref, hs_ref, wih_v, whh_v, sems):
    i = pl.program_id(1)

    @pl.when(i == 0)
    def _():
        cp1 = pltpu.make_async_copy(wih_hbm, wih_v, sems.at[0])
        cp2 = pltpu.make_async_copy(whh_hbm, whh_v, sems.at[1])
        cp1.start()
        cp2.start()
        cp1.wait()
        cp2.wait()

    b4 = b4_ref[...]
    h = jnp.zeros((_NB2, H), BF16)
    cc = jnp.zeros((_NB2, H), F32)
    for s in range(S):
        g = (jnp.dot(x_ref[s].astype(BF16), wih_v[...],
                     preferred_element_type=F32)
             + jnp.dot(h, whh_v[...], preferred_element_type=F32) + b4)
        ig = jax.nn.sigmoid(g[:, :H])
        fg = jax.nn.sigmoid(g[:, H:2 * H])
        gg = jnp.tanh(g[:, 2 * H:3 * H])
        og = jax.nn.sigmoid(g[:, 3 * H:])
        cc = fg * cc + ig * gg
        hf = og * jnp.tanh(cc)
        hs_ref[s] = hf
        h = hf.astype(BF16)


def _lstm_call(x, Wih, Whh, b4):
    return pl.pallas_call(
        _lstm_body,
        grid=(2, _G2 // 2),
        in_specs=[
            pl.BlockSpec((S, _NB2, H), lambda c, i: (0, c * (_G2 // 2) + i, 0)),
            pl.BlockSpec(memory_space=pl.ANY),
            pl.BlockSpec(memory_space=pl.ANY),
            pl.BlockSpec((1, 4 * H), lambda c, i: (0, 0)),
        ],
        out_specs=pl.BlockSpec(
            (S, _NB2, H), lambda c, i: (0, c * (_G2 // 2) + i, 0)),
        scratch_shapes=[
            pltpu.VMEM((H, 4 * H), BF16),
            pltpu.VMEM((H, 4 * H), BF16),
            pltpu.SemaphoreType.DMA((2,)),
        ],
        out_shape=jax.ShapeDtypeStruct((S, N, H), F32),
        compiler_params=pltpu.CompilerParams(
            dimension_semantics=(pltpu.ARBITRARY, pltpu.ARBITRARY),
            vmem_limit_bytes=54 * 1024 * 1024,
        ),
        name="k2_lstm",
    )(x, Wih, Whh, b4)


_KB3 = 512
_G3 = 20


def _ent_body(ents_ref, we_ref, out_ref):
    kc = pl.program_id(0)
    ki = pl.program_id(1)
    kabs = kc * _G3 + ki
    m = ents_ref[...].reshape(N, _KB3).astype(BF16)
    ones = jnp.ones((_KB3, 128), BF16)

    def accum(w_val):
        part = jnp.dot(m, jnp.concatenate([w_val, ones], axis=-1),
                       preferred_element_type=F32)

        @pl.when(ki == 0)
        def _():
            out_ref[0] = part

        @pl.when(ki > 0)
        def _():
            out_ref[0] = out_ref[0] + part

    n_blocks = 2 * _G3
    edge_valid = ENT - (n_blocks - 1) * _KB3

    @pl.when(kabs < n_blocks - 1)
    def _():
        accum(we_ref[...])

    @pl.when(kabs == n_blocks - 1)
    def _():
        row = lax.broadcasted_iota(jnp.int32, (_KB3, H), 0)
        accum(jnp.where(row < edge_valid, we_ref[...], 0))


def _ent_call(ents2, ent_E):
    return pl.pallas_call(
        _ent_body,
        grid=(2, _G3),
        in_specs=[
            pl.BlockSpec((B, P, _KB3), lambda kc, ki: (0, 0, kc * _G3 + ki)),
            pl.BlockSpec((_KB3, H), lambda kc, ki: (kc * _G3 + ki, 0)),
        ],
        out_specs=pl.BlockSpec((1, N, H + 128), lambda kc, ki: (kc, 0, 0)),
        out_shape=jax.ShapeDtypeStruct((2, N, H + 128), F32),
        compiler_params=pltpu.CompilerParams(
            dimension_semantics=(pltpu.ARBITRARY, pltpu.ARBITRARY),
            vmem_limit_bytes=54 * 1024 * 1024,
        ),
        name="k3_entities",
    )(ents2, ent_E)


_NB4 = 128
_G4 = N // _NB4


def _rec_body(hs_ref, pe_ref, w1a_ref, w1b_ref, w2_ref, b1_ref, b2_ref,
              tree_ref, ep_ref):
    b1 = b1_ref[...]
    b2 = b2_ref[...]
    e = hs_ref[0]
    for t in range(1, S):
        a = (jnp.dot(e.astype(BF16), w1a_ref[...],
                     preferred_element_type=F32)
             + jnp.dot(hs_ref[t].astype(BF16), w1b_ref[...],
                       preferred_element_type=F32)
             + b1)
        a = jnp.maximum(a, 0.01 * a)
        e = jnp.dot(a.astype(BF16), w2_ref[...],
                    preferred_element_type=F32) + b2
        e = jnp.maximum(e, 0.01 * e)
    tree_ref[...] = jnp.mean(e.reshape(_NB4 // P, P, H), axis=1)

    pe = pe_ref[0] + pe_ref[1]
    cnt = pe[:, H:]
    rcp = 1.0 / jnp.maximum(cnt, 1.0)
    ep = pe[:, :H] * pltpu.repeat(rcp, 8, axis=1)
    ep_ref[...] = jnp.mean(ep.reshape(_NB4 // P, P, H), axis=1)


def _rec_call(hs, pe_parts, Wm1a, Wm1b, Wm2, bm1, bm2):
    nb = _NB4 // P
    return pl.pallas_call(
        _rec_body,
        grid=(2, _G4 // 2),
        in_specs=[
            pl.BlockSpec((S, _NB4, H), lambda c, i: (0, c * (_G4 // 2) + i, 0)),
            pl.BlockSpec((2, _NB4, H + 128),
                         lambda c, i: (0, c * (_G4 // 2) + i, 0)),
            pl.BlockSpec((H, H), lambda c, i: (0, 0)),
            pl.BlockSpec((H, H), lambda c, i: (0, 0)),
            pl.BlockSpec((H, H), lambda c, i: (0, 0)),
            pl.BlockSpec((1, H), lambda c, i: (0, 0)),
            pl.BlockSpec((1, H), lambda c, i: (0, 0)),
        ],
        out_specs=[
            pl.BlockSpec((nb, H), lambda c, i: (c * (_G4 // 2) + i, 0)),
            pl.BlockSpec((nb, H), lambda c, i: (c * (_G4 // 2) + i, 0)),
        ],
        out_shape=[
            jax.ShapeDtypeStruct((B, H), F32),
            jax.ShapeDtypeStruct((B, H), F32),
        ],
        compiler_params=pltpu.CompilerParams(
            dimension_semantics=(pltpu.ARBITRARY, pltpu.ARBITRARY),
            vmem_limit_bytes=54 * 1024 * 1024,
        ),
        name="k4_rec",
    )(hs, pe_parts, Wm1a, Wm1b, Wm2, bm1, bm2)


def _ln(x, g, b):
    m = jnp.mean(x, axis=-1, keepdims=True)
    v = jnp.mean((x - m) ** 2, axis=-1, keepdims=True)
    return (x - m) * lax.rsqrt(v + 1e-5) * g + b


def _head_body(tree_ref, ep_ref, wf1, bf1_, g1e_, b1e_, g2e_, b2e_, wf2, bf2_,
               wv, bv_, wo, bo_, g1t_, b1t_, wp1, bp1_, wp2, bp2_,
               loss_ref, p_ref):
    e = _ln(ep_ref[...], g1e_[...], b1e_[...])
    e = jnp.maximum(
        jnp.dot(e, wf1[...], preferred_element_type=F32) + bf1_[...], 0.0
    ) * BN_SCALE
    e = _ln(e, g2e_[...], b2e_[...])
    e = jnp.maximum(
        jnp.dot(e, wf2[...], preferred_element_type=F32) + bf2_[...], 0.0
    ) * BN_SCALE
    v = jnp.dot(e, wv[...], preferred_element_type=F32) + bv_[...]
    attn = jnp.dot(v, wo[...], preferred_element_type=F32) + bo_[...]
    tree = _ln(tree_ref[...] + attn, g1t_[...], b1t_[...])
    ph = jnp.maximum(
        jnp.dot(tree, wp1[...], preferred_element_type=F32) + bp1_[...], 0.0)
    p = jnp.dot(ph, wp2[...], preferred_element_type=F32) + bp2_[...]
    nrm = jnp.sqrt(jnp.sum(p * p, axis=-1, keepdims=True))
    p = p / jnp.maximum(nrm, 1e-12)
    p_ref[...] = p

    sim = lax.dot_general(p, p, (((1,), (1,)), ((), ())),
                          preferred_element_type=F32)
    r = lax.broadcasted_iota(jnp.int32, (B, B), 0)
    cdx = lax.broadcasted_iota(jnp.int32, (B, B), 1)
    sim = (sim - (r == cdx).astype(F32)) * (1.0 / TAO)
    mx = jnp.max(sim, axis=-1, keepdims=True)
    lse = jnp.log(jnp.sum(jnp.exp(sim - mx), axis=-1, keepdims=True)) + mx
    ysel = (cdx == jnp.bitwise_xor(r, 1)).astype(F32)
    picked = jnp.sum(sim * ysel, axis=-1, keepdims=True) - lse
    loss_ref[...] = -jnp.sum(picked, axis=0, keepdims=True) * (1.0 / B)


def _head_call(tree, epre, Wf1, bf1, g1e, b1e, g2e, b2e, Wf2, bf2,
               Wv, bv, Wo, bo, g1t, b1t, Wp1, bp1, Wp2, bp2):
    return pl.pallas_call(
        _head_body,
        out_shape=[
            jax.ShapeDtypeStruct((1, 1), F32),
            jax.ShapeDtypeStruct((B, H), F32),
        ],
        compiler_params=pltpu.CompilerParams(
            vmem_limit_bytes=54 * 1024 * 1024,
        ),
        name="k5_head",
    )(tree, epre, Wf1, bf1, g1e, b1e, g2e, b2e, Wf2, bf2,
      Wv, bv, Wo, bo, g1t, b1t, Wp1, bp1, Wp2, bp2)


def kernel(rel_E, Wih, Whh, bih, bhh, Wm1, bm1, Wm2, bm2,
           ent_E, Wf1, bf1, Wf2, bf2, g1e, b1e, g2e, b2e,
           Wq, bq, Wk, bk, Wv, bv, Wo, bo, g1t, b1t,
           Wp1, bp1, Wp2, bp2, paths, entities):
    r2 = lambda a: a.reshape(1, -1)
    idx8 = paths.reshape(-1).astype(jnp.int32) * 8
    rel2 = rel_E.reshape(REL * 8, 128)
    b4 = r2(bih + bhh)

    bf = lambda a: a.astype(jnp.bfloat16)
    x = _gather_call(idx8, rel2)
    hs = _lstm_call(x, bf(Wih), bf(Whh), b4)
    pe_parts = _ent_call(entities, bf(ent_E))
    tree, epre = _rec_call(hs, pe_parts, bf(Wm1[:H]), bf(Wm1[H:]), bf(Wm2),
                           r2(bm1), r2(bm2))
    loss2, p = _head_call(tree, epre, Wf1, r2(bf1), r2(g1e), r2(b1e),
                          r2(g2e), r2(b2e), Wf2, r2(bf2), Wv, r2(bv),
                          Wo, r2(bo), r2(g1t), r2(b1t), Wp1, r2(bp1),
                          Wp2, r2(bp2))
    return loss2.reshape(()), p


# speedup vs baseline: 1.5555x; 1.0116x over previous
import jax
import jax.numpy as jnp
import numpy as np
from jax import lax
from jax.experimental import pallas as pl
from jax.experimental.pallas import tpu as pltpu

H = 1024
B = 128
P = 16
S = 12
REL = 5000
ENT = 20000
N = B * P
TAO = 0.5
BN_SCALE = 1.0 / float(np.sqrt(1.0 + 1e-5))
F32 = jnp.float32

_NB1 = 128
_G1 = N // _NB1
_STR = 136


def _gather_body(idx_ref, rel_hbm, x_ref, rel_v, tile_a, tile_b, sem):
    c = pl.program_id(0)
    i = pl.program_id(1)

    @pl.when(i == 0)
    def _():
        cp = pltpu.make_async_copy(rel_hbm, rel_v, sem)
        cp.start()
        cp.wait()

    base = (c * (_G1 // 2) + i) * (_NB1 * S)
    for s in range(S):
        tile = tile_a if (s % 2 == 0) else tile_b
        for mi in range(_NB1):
            r8 = pl.multiple_of(idx_ref[base + mi * S + s], 8)
            tile[mi: mi + 8 * _STR: _STR, :] = rel_v[pl.ds(r8, 8), :]
        xs = jnp.concatenate(
            [tile[pl.ds(j * _STR, _NB1), :] for j in range(8)], axis=-1)
        x_ref[s] = xs.astype(jnp.bfloat16)


def _gather_call(idx8, rel2):
    return pl.pallas_call(
        _gather_body,
        grid_spec=pltpu.PrefetchScalarGridSpec(
            num_scalar_prefetch=1,
            grid=(2, _G1 // 2),
            in_specs=[pl.BlockSpec(memory_space=pl.ANY)],
            out_specs=pl.BlockSpec(
                (S, _NB1, H), lambda c, i, idx: (0, c * (_G1 // 2) + i, 0)),
            scratch_shapes=[
                pltpu.VMEM((REL * 8, 128), F32),
                pltpu.VMEM((8 * _STR, 128), F32),
                pltpu.VMEM((8 * _STR, 128), F32),
                pltpu.SemaphoreType.DMA,
            ],
        ),
        out_shape=jax.ShapeDtypeStruct((S, N, H), jnp.bfloat16),
        compiler_params=pltpu.CompilerParams(
            dimension_semantics=(pltpu.ARBITRARY, pltpu.ARBITRARY),
            vmem_limit_bytes=52 * 1024 * 1024,
        ),
        name="k1_gather",
    )(idx8, rel2)


_NB2 = 128
_G2 = N // _NB2
BF16 = jnp.bfloat16


def _lstm_body(x_ref, wih_hbm, whh_hbm, b4_ref, hs_ref, wih_v, whh_v, sems):
    i = pl.program_id(1)

    @pl.when(i == 0)
    def _():
        cp1 = pltpu.make_async_copy(wih_hbm, wih_v, sems.at[0])
        cp2 = pltpu.make_async_copy(whh_hbm, whh_v, sems.at[1])
        cp1.start()
        cp2.start()
        cp1.wait()
        cp2.wait()

    b4 = b4_ref[...]
    h = jnp.zeros((_NB2, H), BF16)
    cc = jnp.zeros((_NB2, H), F32)
    for s in range(S):
        g = (jnp.dot(x_ref[s], wih_v[...], preferred_element_type=F32)
             + jnp.dot(h, whh_v[...], preferred_element_type=F32) + b4)
        ig = jax.nn.sigmoid(g[:, :H])
        fg = jax.nn.sigmoid(g[:, H:2 * H])
        gg = jnp.tanh(g[:, 2 * H:3 * H])
        og = jax.nn.sigmoid(g[:, 3 * H:])
        cc = fg * cc + ig * gg
        h = (og * jnp.tanh(cc)).astype(BF16)
        hs_ref[s] = h


def _lstm_call(x, Wih, Whh, b4):
    return pl.pallas_call(
        _lstm_body,
        grid=(2, _G2 // 2),
        in_specs=[
            pl.BlockSpec((S, _NB2, H), lambda c, i: (0, c * (_G2 // 2) + i, 0)),
            pl.BlockSpec(memory_space=pl.ANY),
            pl.BlockSpec(memory_space=pl.ANY),
            pl.BlockSpec((1, 4 * H), lambda c, i: (0, 0)),
        ],
        out_specs=pl.BlockSpec(
            (S, _NB2, H), lambda c, i: (0, c * (_G2 // 2) + i, 0)),
        scratch_shapes=[
            pltpu.VMEM((H, 4 * H), BF16),
            pltpu.VMEM((H, 4 * H), BF16),
            pltpu.SemaphoreType.DMA((2,)),
        ],
        out_shape=jax.ShapeDtypeStruct((S, N, H), BF16),
        compiler_params=pltpu.CompilerParams(
            dimension_semantics=(pltpu.ARBITRARY, pltpu.ARBITRARY),
            vmem_limit_bytes=54 * 1024 * 1024,
        ),
        name="k2_lstm",
    )(x, Wih, Whh, b4)


_KB3 = 512
_G3 = 20


def _ent_body(ents_ref, we_ref, out_ref):
    kc = pl.program_id(0)
    ki = pl.program_id(1)
    kabs = kc * _G3 + ki
    m = ents_ref[...].reshape(N, _KB3).astype(BF16)
    ones = jnp.ones((_KB3, 128), BF16)

    def accum(w_val):
        part = jnp.dot(m, jnp.concatenate([w_val, ones], axis=-1),
                       preferred_element_type=F32)

        @pl.when(ki == 0)
        def _():
            out_ref[0] = part

        @pl.when(ki > 0)
        def _():
            out_ref[0] = out_ref[0] + part

    n_blocks = 2 * _G3
    edge_valid = ENT - (n_blocks - 1) * _KB3

    @pl.when(kabs < n_blocks - 1)
    def _():
        accum(we_ref[...])

    @pl.when(kabs == n_blocks - 1)
    def _():
        row = lax.broadcasted_iota(jnp.int32, (_KB3, H), 0)
        accum(jnp.where(row < edge_valid, we_ref[...], 0))


def _ent_call(ents2, ent_E):
    return pl.pallas_call(
        _ent_body,
        grid=(2, _G3),
        in_specs=[
            pl.BlockSpec((B, P, _KB3), lambda kc, ki: (0, 0, kc * _G3 + ki)),
            pl.BlockSpec((_KB3, H), lambda kc, ki: (kc * _G3 + ki, 0)),
        ],
        out_specs=pl.BlockSpec((1, N, H + 128), lambda kc, ki: (kc, 0, 0)),
        out_shape=jax.ShapeDtypeStruct((2, N, H + 128), F32),
        compiler_params=pltpu.CompilerParams(
            dimension_semantics=(pltpu.ARBITRARY, pltpu.ARBITRARY),
            vmem_limit_bytes=54 * 1024 * 1024,
        ),
        name="k3_entities",
    )(ents2, ent_E)


_NB4 = 128
_G4 = N // _NB4


def _rec_body(hs_ref, pe_ref, w1a_ref, w1b_ref, w2_ref, b1_ref, b2_ref,
              tree_ref, ep_ref):
    b1 = b1_ref[...]
    b2 = b2_ref[...]
    e_b = hs_ref[0]
    e = e_b.astype(F32)
    for t in range(1, S):
        a = (jnp.dot(e_b, w1a_ref[...], preferred_element_type=F32)
             + jnp.dot(hs_ref[t], w1b_ref[...], preferred_element_type=F32)
             + b1)
        a = jnp.maximum(a, 0.01 * a)
        e = jnp.dot(a.astype(BF16), w2_ref[...],
                    preferred_element_type=F32) + b2
        e = jnp.maximum(e, 0.01 * e)
        e_b = e.astype(BF16)
    tree_ref[...] = jnp.mean(e.reshape(_NB4 // P, P, H), axis=1)

    pe = pe_ref[0] + pe_ref[1]
    cnt = pe[:, H:]
    rcp = 1.0 / jnp.maximum(cnt, 1.0)
    ep = pe[:, :H] * pltpu.repeat(rcp, 8, axis=1)
    ep_ref[...] = jnp.mean(ep.reshape(_NB4 // P, P, H), axis=1)


def _rec_call(hs, pe_parts, Wm1a, Wm1b, Wm2, bm1, bm2):
    nb = _NB4 // P
    return pl.pallas_call(
        _rec_body,
        grid=(2, _G4 // 2),
        in_specs=[
            pl.BlockSpec((S, _NB4, H), lambda c, i: (0, c * (_G4 // 2) + i, 0)),
            pl.BlockSpec((2, _NB4, H + 128),
                         lambda c, i: (0, c * (_G4 // 2) + i, 0)),
            pl.BlockSpec((H, H), lambda c, i: (0, 0)),
            pl.BlockSpec((H, H), lambda c, i: (0, 0)),
            pl.BlockSpec((H, H), lambda c, i: (0, 0)),
            pl.BlockSpec((1, H), lambda c, i: (0, 0)),
            pl.BlockSpec((1, H), lambda c, i: (0, 0)),
        ],
        out_specs=[
            pl.BlockSpec((nb, H), lambda c, i: (c * (_G4 // 2) + i, 0)),
            pl.BlockSpec((nb, H), lambda c, i: (c * (_G4 // 2) + i, 0)),
        ],
        out_shape=[
            jax.ShapeDtypeStruct((B, H), F32),
            jax.ShapeDtypeStruct((B, H), F32),
        ],
        compiler_params=pltpu.CompilerParams(
            dimension_semantics=(pltpu.ARBITRARY, pltpu.ARBITRARY),
            vmem_limit_bytes=54 * 1024 * 1024,
        ),
        name="k4_rec",
    )(hs, pe_parts, Wm1a, Wm1b, Wm2, bm1, bm2)


def _ln(x, g, b):
    m = jnp.mean(x, axis=-1, keepdims=True)
    v = jnp.mean((x - m) ** 2, axis=-1, keepdims=True)
    return (x - m) * lax.rsqrt(v + 1e-5) * g + b


def _head_body(tree_ref, ep_ref, wf1, bf1_, g1e_, b1e_, g2e_, b2e_, wf2, bf2_,
               wv, bv_, wo, bo_, g1t_, b1t_, wp1, bp1_, wp2, bp2_,
               loss_ref, p_ref):
    e = _ln(ep_ref[...], g1e_[...], b1e_[...])
    e = jnp.maximum(
        jnp.dot(e, wf1[...], preferred_element_type=F32) + bf1_[...], 0.0
    ) * BN_SCALE
    e = _ln(e, g2e_[...], b2e_[...])
    e = jnp.maximum(
        jnp.dot(e, wf2[...], preferred_element_type=F32) + bf2_[...], 0.0
    ) * BN_SCALE
    v = jnp.dot(e, wv[...], preferred_element_type=F32) + bv_[...]
    attn = jnp.dot(v, wo[...], preferred_element_type=F32) + bo_[...]
    tree = _ln(tree_ref[...] + attn, g1t_[...], b1t_[...])
    ph = jnp.maximum(
        jnp.dot(tree, wp1[...], preferred_element_type=F32) + bp1_[...], 0.0)
    p = jnp.dot(ph, wp2[...], preferred_element_type=F32) + bp2_[...]
    nrm = jnp.sqrt(jnp.sum(p * p, axis=-1, keepdims=True))
    p = p / jnp.maximum(nrm, 1e-12)
    p_ref[...] = p

    sim = lax.dot_general(p, p, (((1,), (1,)), ((), ())),
                          preferred_element_type=F32)
    r = lax.broadcasted_iota(jnp.int32, (B, B), 0)
    cdx = lax.broadcasted_iota(jnp.int32, (B, B), 1)
    sim = (sim - (r == cdx).astype(F32)) * (1.0 / TAO)
    mx = jnp.max(sim, axis=-1, keepdims=True)
    lse = jnp.log(jnp.sum(jnp.exp(sim - mx), axis=-1, keepdims=True)) + mx
    ysel = (cdx == jnp.bitwise_xor(r, 1)).astype(F32)
    picked = jnp.sum(sim * ysel, axis=-1, keepdims=True) - lse
    loss_ref[...] = -jnp.sum(picked, axis=0, keepdims=True) * (1.0 / B)


def _head_call(tree, epre, Wf1, bf1, g1e, b1e, g2e, b2e, Wf2, bf2,
               Wv, bv, Wo, bo, g1t, b1t, Wp1, bp1, Wp2, bp2):
    return pl.pallas_call(
        _head_body,
        out_shape=[
            jax.ShapeDtypeStruct((1, 1), F32),
            jax.ShapeDtypeStruct((B, H), F32),
        ],
        compiler_params=pltpu.CompilerParams(
            vmem_limit_bytes=54 * 1024 * 1024,
        ),
        name="k5_head",
    )(tree, epre, Wf1, bf1, g1e, b1e, g2e, b2e, Wf2, bf2,
      Wv, bv, Wo, bo, g1t, b1t, Wp1, bp1, Wp2, bp2)


def kernel(rel_E, Wih, Whh, bih, bhh, Wm1, bm1, Wm2, bm2,
           ent_E, Wf1, bf1, Wf2, bf2, g1e, b1e, g2e, b2e,
           Wq, bq, Wk, bk, Wv, bv, Wo, bo, g1t, b1t,
           Wp1, bp1, Wp2, bp2, paths, entities):
    r2 = lambda a: a.reshape(1, -1)
    idx8 = paths.reshape(-1).astype(jnp.int32) * 8
    rel2 = rel_E.reshape(REL * 8, 128)
    b4 = r2(bih + bhh)

    bf = lambda a: a.astype(jnp.bfloat16)
    x = _gather_call(idx8, rel2)
    hs = _lstm_call(x, bf(Wih), bf(Whh), b4)
    pe_parts = _ent_call(entities, bf(ent_E))
    tree, epre = _rec_call(hs, pe_parts, bf(Wm1[:H]), bf(Wm1[H:]), bf(Wm2),
                           r2(bm1), r2(bm2))
    loss2, p = _head_call(tree, epre, Wf1, r2(bf1), r2(g1e), r2(b1e),
                          r2(g2e), r2(b2e), Wf2, r2(bf2), Wv, r2(bv),
                          Wo, r2(bo), r2(g1t), r2(b1t), Wp1, r2(bp1),
                          Wp2, r2(bp2))
    return loss2.reshape(()), p
